# Initial kernel scaffold; baseline (speedup 1.0000x reference)
import jax
import jax.numpy as jnp
from jax.experimental import pallas as pl


def kernel(input_ids, weight):
    raise NotImplementedError("write your pallas kernel here")


# speedup vs baseline: 3.3193x; 3.3193x over previous
import functools

import jax
import jax.numpy as jnp
from jax import lax
from jax.experimental import pallas as pl
from jax.experimental.pallas import tpu as pltpu
from jax.experimental.pallas import tpu_sc as plsc

_VOCAB = 100000
_D = 128
_B_TOT = 4096 * 50
_NC = 2
_NS = 16
_NW = _NC * _NS
_PER_W = _B_TOT // _NW
_CHUNK = 128
_CPW = _PER_W // _CHUNK


def _embed_body(idx_hbm, table_hbm, out_hbm, idx_v, buf0, buf1, sem0, sem1):
    wid = lax.axis_index("s") * _NC + lax.axis_index("c")
    pltpu.sync_copy(idx_hbm.at[wid], idx_v)
    base = wid * _PER_W

    pltpu.async_copy(table_hbm.at[idx_v.at[0]], buf0, sem0)
    pltpu.async_copy(table_hbm.at[idx_v.at[1]], buf1, sem1)

    @pl.loop(0, _CPW // 2)
    def _(i):
        jj = i * 2
        pltpu.make_async_copy(table_hbm.at[idx_v.at[0]], buf0, sem0).wait()
        pltpu.sync_copy(buf0, out_hbm.at[pl.ds(base + jj * _CHUNK, _CHUNK)])

        @pl.when(jj + 2 < _CPW)
        def _():
            pltpu.async_copy(table_hbm.at[idx_v.at[jj + 2]], buf0, sem0)

        pltpu.make_async_copy(table_hbm.at[idx_v.at[1]], buf1, sem1).wait()
        pltpu.sync_copy(buf1, out_hbm.at[pl.ds(base + (jj + 1) * _CHUNK, _CHUNK)])

        @pl.when(jj + 3 < _CPW)
        def _():
            pltpu.async_copy(table_hbm.at[idx_v.at[jj + 3]], buf1, sem1)


@jax.jit
def kernel(input_ids, weight):
    batch, hist = input_ids.shape
    idx3d = input_ids.astype(jnp.int32).reshape(_NW, _CPW, _CHUNK)
    gathered = pl.kernel(
        _embed_body,
        out_type=jax.ShapeDtypeStruct((_B_TOT, _D), jnp.float32),
        mesh=plsc.VectorSubcoreMesh(core_axis_name="c", subcore_axis_name="s"),
        scratch_types=[
            pltpu.VMEM((_CPW, _CHUNK), jnp.int32),
            pltpu.VMEM((_CHUNK, _D), jnp.float32),
            pltpu.VMEM((_CHUNK, _D), jnp.float32),
            pltpu.SemaphoreType.DMA,
            pltpu.SemaphoreType.DMA,
        ],
    )(idx3d, weight)
    return gathered.reshape(batch, hist, _D)


# speedup vs baseline: 5.1093x; 1.5393x over previous
import functools

import jax
import jax.numpy as jnp
from jax import lax
from jax.experimental import pallas as pl
from jax.experimental.pallas import tpu as pltpu
from jax.experimental.pallas import tpu_sc as plsc

_VOCAB = 100000
_D = 128
_BATCH = 4096
_HIST = 50
_NC = 2
_NS = 16
_NW = _NC * _NS
_BPW = _BATCH // _NW


def _embed_body(idx_hbm, table_hbm, out_hbm, idx_v, buf0, buf1, sem0, sem1):
    wid = lax.axis_index("s") * _NC + lax.axis_index("c")
    base = wid * _BPW
    pltpu.sync_copy(idx_hbm.at[pl.ds(base, _BPW)], idx_v)

    pltpu.async_copy(table_hbm.at[idx_v.at[0]], buf0, sem0)
    pltpu.async_copy(table_hbm.at[idx_v.at[1]], buf1, sem1)

    @pl.loop(0, _BPW // 2)
    def _(i):
        k = i * 2
        pltpu.make_async_copy(table_hbm.at[idx_v.at[0]], buf0, sem0).wait()
        pltpu.sync_copy(buf0, out_hbm.at[base + k])

        @pl.when(k + 2 < _BPW)
        def _():
            pltpu.async_copy(table_hbm.at[idx_v.at[k + 2]], buf0, sem0)

        pltpu.make_async_copy(table_hbm.at[idx_v.at[1]], buf1, sem1).wait()
        pltpu.sync_copy(buf1, out_hbm.at[base + k + 1])

        @pl.when(k + 3 < _BPW)
        def _():
            pltpu.async_copy(table_hbm.at[idx_v.at[k + 3]], buf1, sem1)


@jax.jit
def kernel(input_ids, weight):
    idx = input_ids.astype(jnp.int32)
    return pl.kernel(
        _embed_body,
        out_type=jax.ShapeDtypeStruct((_BATCH, _HIST, _D), jnp.float32),
        mesh=plsc.VectorSubcoreMesh(core_axis_name="c", subcore_axis_name="s"),
        scratch_types=[
            pltpu.VMEM((_BPW, _HIST), jnp.int32),
            pltpu.VMEM((_HIST, _D), jnp.float32),
            pltpu.VMEM((_HIST, _D), jnp.float32),
            pltpu.SemaphoreType.DMA,
            pltpu.SemaphoreType.DMA,
        ],
    )(idx, weight)


# speedup vs baseline: 5.9262x; 1.1599x over previous
import functools

import jax
import jax.numpy as jnp
from jax import lax
from jax.experimental import pallas as pl
from jax.experimental.pallas import tpu as pltpu
from jax.experimental.pallas import tpu_sc as plsc

_VOCAB = 100000
_D = 128
_BATCH = 4096
_HIST = 50
_NC = 2
_NS = 16
_NW = _NC * _NS
_BPW = _BATCH // _NW
_G = 4
_NGRP = _BPW // _G
_NSETS = 4


def _embed_body(idx_hbm, table_hbm, out_hbm, idx_v, bufs,
                sg0, sg1, sg2, sg3, sw0, sw1, sw2, sw3):
    sg = (sg0, sg1, sg2, sg3)
    sw = (sw0, sw1, sw2, sw3)
    wid = lax.axis_index("s") * _NC + lax.axis_index("c")
    base = wid * _BPW
    pltpu.sync_copy(idx_hbm.at[pl.ds(base, _BPW)], idx_v)

    def fire_gathers(m, s):
        for j in range(_G):
            pltpu.async_copy(table_hbm.at[idx_v.at[m * _G + j]],
                             bufs.at[s * _G + j], sg[s])

    def drain_gathers(s):
        for j in range(_G):
            pltpu.make_async_copy(table_hbm.at[idx_v.at[0]],
                                  bufs.at[s * _G + j], sg[s]).wait()

    def fire_writes(m, s):
        for j in range(_G):
            pltpu.async_copy(bufs.at[s * _G + j],
                             out_hbm.at[base + m * _G + j], sw[s])

    def drain_writes(s):
        for j in range(_G):
            pltpu.make_async_copy(bufs.at[s * _G + j],
                                  out_hbm.at[base + j], sw[s]).wait()

    fire_gathers(0, 0)
    fire_gathers(1, 1)

    @pl.loop(0, _NGRP // _NSETS)
    def _(i):
        for p in range(_NSETS):
            m = i * _NSETS + p
            drain_gathers(p)
            fire_writes(m, p)
            s_next = (p + 2) % _NSETS
            @pl.when(m >= 2)
            def _():
                drain_writes(s_next)

            @pl.when(m + 2 < _NGRP)
            def _():
                fire_gathers(m + 2, s_next)

    drain_writes(2)
    drain_writes(3)


@jax.jit
def kernel(input_ids, weight):
    idx = input_ids.astype(jnp.int32)
    return pl.kernel(
        _embed_body,
        out_type=jax.ShapeDtypeStruct((_BATCH, _HIST, _D), jnp.float32),
        mesh=plsc.VectorSubcoreMesh(core_axis_name="c", subcore_axis_name="s"),
        scratch_types=[
            pltpu.VMEM((_BPW, _HIST), jnp.int32),
            pltpu.VMEM((_NSETS * _G, _HIST, _D), jnp.float32),
            pltpu.SemaphoreType.DMA,
            pltpu.SemaphoreType.DMA,
            pltpu.SemaphoreType.DMA,
            pltpu.SemaphoreType.DMA,
            pltpu.SemaphoreType.DMA,
            pltpu.SemaphoreType.DMA,
            pltpu.SemaphoreType.DMA,
            pltpu.SemaphoreType.DMA,
        ],
    )(idx, weight)


# speedup vs baseline: 5.9341x; 1.0013x over previous
import functools

import jax
import jax.numpy as jnp
from jax import lax
from jax.experimental import pallas as pl
from jax.experimental.pallas import tpu as pltpu
from jax.experimental.pallas import tpu_sc as plsc

_VOCAB = 100000
_D = 128
_BATCH = 4096
_HIST = 50
_NC = 2
_NS = 16
_NW = _NC * _NS
_BPW = _BATCH // _NW
_G = 4
_NGRP = _BPW // _G
_NSETS = 4


def _embed_body(idx_hbm, table_hbm, out_hbm, idx_v, bufs,
                sg0, sg1, sg2, sg3, sw0, sw1, sw2, sw3):
    sg = (sg0, sg1, sg2, sg3)
    sw = (sw0, sw1, sw2, sw3)
    wid = lax.axis_index("s") * _NC + lax.axis_index("c")
    base = wid * _BPW
    pltpu.sync_copy(idx_hbm.at[pl.ds(base, _BPW)], idx_v)

    def fire_gathers(m, s):
        for j in range(_G):
            pltpu.async_copy(table_hbm.at[idx_v.at[m * _G + j]],
                             bufs.at[s * _G + j], sg[s])

    def drain_gathers(s):
        for j in range(_G):
            pltpu.make_async_copy(table_hbm.at[idx_v.at[0]],
                                  bufs.at[s * _G + j], sg[s]).wait()

    def fire_writes(m, s):
        for j in range(_G):
            pltpu.async_copy(bufs.at[s * _G + j],
                             out_hbm.at[base + m * _G + j], sw[s])

    def drain_writes(s):
        for j in range(_G):
            pltpu.make_async_copy(bufs.at[s * _G + j],
                                  out_hbm.at[base + j], sw[s]).wait()

    fire_gathers(0, 0)
    fire_gathers(1, 1)

    @pl.loop(0, _NGRP // _NSETS)
    def _(i):
        for p in range(_NSETS):
            m = i * _NSETS + p
            drain_gathers(p)
            fire_writes(m, p)
            s_next = (p + 2) % _NSETS
            @pl.when(m >= 2)
            def _():
                drain_writes(s_next)

            @pl.when(m + 2 < _NGRP)
            def _():
                fire_gathers(m + 2, s_next)

    drain_writes(2)
    drain_writes(3)


@jax.jit
def kernel(input_ids, weight):
    idx = input_ids.astype(jnp.int32)
    return pl.kernel(
        _embed_body,
        out_type=jax.ShapeDtypeStruct((_BATCH, _HIST, _D), jnp.float32),
        mesh=plsc.VectorSubcoreMesh(core_axis_name="c", subcore_axis_name="s"),
        compiler_params=pltpu.CompilerParams(use_tc_tiling_on_sc=True),
        scratch_types=[
            pltpu.VMEM((_BPW, _HIST), jnp.int32),
            pltpu.VMEM((_NSETS * _G, _HIST, _D), jnp.float32),
            pltpu.SemaphoreType.DMA,
            pltpu.SemaphoreType.DMA,
            pltpu.SemaphoreType.DMA,
            pltpu.SemaphoreType.DMA,
            pltpu.SemaphoreType.DMA,
            pltpu.SemaphoreType.DMA,
            pltpu.SemaphoreType.DMA,
            pltpu.SemaphoreType.DMA,
        ],
    )(idx, weight)


# speedup vs baseline: 5.9437x; 1.0016x over previous
import functools

import jax
import jax.numpy as jnp
from jax import lax
from jax.experimental import pallas as pl
from jax.experimental.pallas import tpu as pltpu
from jax.experimental.pallas import tpu_sc as plsc

_VOCAB = 100000
_D = 128
_BATCH = 4096
_HIST = 50
_NC = 2
_NS = 16
_NW = _NC * _NS
_BPW = _BATCH // _NW
_G = 4
_NGRP = _BPW // _G
_NSETS = 4


def _embed_body(idx_hbm, table_hbm, out_hbm, idx_v, bufs,
                sg0, sg1, sg2, sg3, sw0, sw1, sw2, sw3):
    sg = (sg0, sg1, sg2, sg3)
    sw = (sw0, sw1, sw2, sw3)
    wid = lax.axis_index("s") * _NC + lax.axis_index("c")
    base = wid * _BPW
    pltpu.sync_copy(idx_hbm.at[pl.ds(base, _BPW)], idx_v)

    def fire_gathers(m, s):
        for j in range(_G):
            pltpu.async_copy(table_hbm.at[idx_v.at[m * _G + j]],
                             bufs.at[s * _G + j], sg[s])

    def drain_gathers(s):
        for j in range(_G):
            pltpu.make_async_copy(table_hbm.at[idx_v.at[0]],
                                  bufs.at[s * _G + j], sg[s]).wait()

    def fire_writes(m, s):
        for j in range(_G):
            pltpu.async_copy(bufs.at[s * _G + j],
                             out_hbm.at[base + m * _G + j], sw[s])

    def drain_writes(s):
        for j in range(_G):
            pltpu.make_async_copy(bufs.at[s * _G + j],
                                  out_hbm.at[base + j], sw[s]).wait()

    fire_gathers(0, 0)
    fire_gathers(1, 1)

    @pl.loop(0, _NGRP // _NSETS)
    def _(i):
        for p in range(_NSETS):
            m = i * _NSETS + p
            drain_gathers(p)
            fire_writes(m, p)
            s_next = (p + 2) % _NSETS
            @pl.when(m >= 2)
            def _():
                drain_writes(s_next)

            @pl.when(m + 2 < _NGRP)
            def _():
                fire_gathers(m + 2, s_next)

    drain_writes(2)
    drain_writes(3)


@jax.jit
def kernel(input_ids, weight):
    idx = input_ids.astype(jnp.int32)
    return pl.kernel(
        _embed_body,
        out_type=jax.ShapeDtypeStruct((_BATCH, _HIST, _D), jnp.float32),
        mesh=plsc.VectorSubcoreMesh(core_axis_name="c", subcore_axis_name="s"),
        compiler_params=pltpu.CompilerParams(
            use_tc_tiling_on_sc=True, needs_layout_passes=True),
        scratch_types=[
            pltpu.VMEM((_BPW, _HIST), jnp.int32),
            pltpu.VMEM((_NSETS * _G, _HIST, _D), jnp.float32),
            pltpu.SemaphoreType.DMA,
            pltpu.SemaphoreType.DMA,
            pltpu.SemaphoreType.DMA,
            pltpu.SemaphoreType.DMA,
            pltpu.SemaphoreType.DMA,
            pltpu.SemaphoreType.DMA,
            pltpu.SemaphoreType.DMA,
            pltpu.SemaphoreType.DMA,
        ],
    )(idx, weight)
